# baseline (speedup 1.0000x reference)
import functools

import jax
import jax.numpy as jnp
from jax.experimental import pallas as pl
from jax.experimental.pallas import tpu as pltpu

_B, _C = 16384, 1000
_BB = 512


def _ce_softmax_kernel(out_ref, y_ref, ypred_ref, loss_ref, acc_ref):
    i = pl.program_id(0)
    n = pl.num_programs(0)

    x = out_ref[...]
    y = y_ref[...]

    m = jnp.max(x, axis=1, keepdims=True)
    e = jnp.exp(x - m)
    s = jnp.sum(e, axis=1, keepdims=True)
    ypred_ref[...] = jnp.clip(e / s, 0.0001, 1.0 - 0.0001)

    lse = m + jnp.log(s)
    part = jnp.sum(jnp.sum(y, axis=1, keepdims=True) * lse) - jnp.sum(y * x)

    @pl.when(i == 0)
    def _init():
        acc_ref[0] = 0.0

    acc_ref[0] += part

    @pl.when(i == n - 1)
    def _fin():
        loss_ref[0] = acc_ref[0] * (1.0 / _B)


@jax.jit
def _run(output, y_labeled):
    grid = (_B // _BB,)
    loss, y_pred = pl.pallas_call(
        _ce_softmax_kernel,
        grid=grid,
        in_specs=[
            pl.BlockSpec((_BB, _C), lambda i: (i, 0)),
            pl.BlockSpec((_BB, _C), lambda i: (i, 0)),
        ],
        out_specs=[
            pl.BlockSpec((_BB, _C), lambda i: (i, 0)),
            pl.BlockSpec(memory_space=pltpu.SMEM),
        ],
        out_shape=[
            jax.ShapeDtypeStruct((_B, _C), jnp.float32),
            jax.ShapeDtypeStruct((1,), jnp.float32),
        ],
        scratch_shapes=[pltpu.SMEM((1,), jnp.float32)],
    )(output, y_labeled)[::-1]
    return loss[0], y_pred


def kernel(iteration, output, y_labeled):
    del iteration
    return _run(output, y_labeled)
